# baseline (speedup 1.0000x reference)
import functools

import jax
import jax.numpy as jnp
from jax import lax
from jax.experimental import pallas as pl
from jax.experimental.pallas import tpu as pltpu
from jax.experimental.pallas import tpu_sc as plsc

BATCH = 4
SEQ = 8192
D_MODEL = 1024

_INFO = plsc.get_sparse_core_info()
NC, NS = _INFO.num_cores, _INFO.num_subcores
NW = NC * NS
ROWS_W = SEQ // NW
CH = 32
NCH = ROWS_W // CH

_mesh = plsc.VectorSubcoreMesh(core_axis_name="c", subcore_axis_name="s")


@functools.partial(
    pl.kernel,
    mesh=_mesh,
    out_type=jax.ShapeDtypeStruct((BATCH, SEQ, D_MODEL), jnp.float32),
    scratch_types=[
        pltpu.VMEM((CH, D_MODEL), jnp.float32),
        pltpu.VMEM((CH, D_MODEL), jnp.float32),
        pltpu.SemaphoreType.DMA,
        pltpu.SemaphoreType.DMA,
        pltpu.SemaphoreType.DMA,
    ],
)
def _sc_broadcast(pos_hbm, out_hbm, buf0, buf1, rsem, wsem0, wsem1):
    wid = lax.axis_index("s") * NC + lax.axis_index("c")
    base = wid * ROWS_W
    bufs = (buf0, buf1)
    wsems = (wsem0, wsem1)
    pending = [[], []]
    for c in range(NCH):
        slot = c % 2
        buf = bufs[slot]
        for w in pending[slot]:
            w.wait()
        pending[slot] = []
        row0 = base + c * CH
        rd = pltpu.async_copy(pos_hbm.at[pl.ds(row0, CH), :], buf, rsem)
        rd.wait()
        for b in range(BATCH):
            pending[slot].append(
                pltpu.async_copy(buf, out_hbm.at[b, pl.ds(row0, CH), :],
                                 wsems[slot]))
    for slot in range(2):
        for w in pending[slot]:
            w.wait()


def kernel(embeddings, pos_table):
    del embeddings
    return _sc_broadcast(pos_table)


# speedup vs baseline: 1.0299x; 1.0299x over previous
import functools

import jax
import jax.numpy as jnp
from jax import lax
from jax.experimental import pallas as pl
from jax.experimental.pallas import tpu as pltpu
from jax.experimental.pallas import tpu_sc as plsc

BATCH = 4
SEQ = 8192
D_MODEL = 1024

_INFO = plsc.get_sparse_core_info()
NC, NS = _INFO.num_cores, _INFO.num_subcores
NW = NC * NS
ROWS_W = SEQ // NW
CH = 32
NCH = ROWS_W // CH

_mesh = plsc.VectorSubcoreMesh(core_axis_name="c", subcore_axis_name="s")


@functools.partial(
    pl.kernel,
    mesh=_mesh,
    out_type=jax.ShapeDtypeStruct((BATCH, SEQ, D_MODEL), jnp.float32),
    scratch_types=[
        pltpu.VMEM((CH, D_MODEL), jnp.float32),
        pltpu.VMEM((CH, D_MODEL), jnp.float32),
        pltpu.SemaphoreType.DMA,
        pltpu.SemaphoreType.DMA,
        pltpu.SemaphoreType.DMA,
    ],
)
def _sc_broadcast(pos_hbm, out_hbm, buf0, buf1, rsem, wsem0, wsem1):
    wid = lax.axis_index("s") * NC + lax.axis_index("c")
    base = wid * ROWS_W
    bufs = (buf0, buf1)
    wsems = (wsem0, wsem1)
    pending = [[], []]
    rd = pltpu.async_copy(pos_hbm.at[pl.ds(base, CH), :], buf0, rsem)
    for c in range(NCH):
        slot = c % 2
        nslot = (c + 1) % 2
        rd.wait()
        if c + 1 < NCH:
            for w in pending[nslot]:
                w.wait()
            pending[nslot] = []
            rd = pltpu.async_copy(
                pos_hbm.at[pl.ds(base + (c + 1) * CH, CH), :], bufs[nslot],
                rsem)
        row0 = base + c * CH
        for b in range(BATCH):
            pending[slot].append(
                pltpu.async_copy(bufs[slot], out_hbm.at[b, pl.ds(row0, CH), :],
                                 wsems[slot]))
    for slot in range(2):
        for w in pending[slot]:
            w.wait()


def kernel(embeddings, pos_table):
    del embeddings
    return _sc_broadcast(pos_table)
